# baseline (speedup 1.0000x reference)
import functools

import jax
import jax.numpy as jnp
from jax import lax
from jax.experimental import pallas as pl
from jax.experimental.pallas import tpu as pltpu
from jax.experimental.pallas import tpu_sc as plsc

_N, _E, _D, _H, _C = 10000, 320000, 128, 16, 16
_NC, _NS = 2, 16
_NW = _NC * _NS
_CHUNK = 128
_NCH = 80
_EPT = _NCH * _CHUNK
_EPAD = _NW * _EPT
_PADN = _N + 16
_NP = 10240
_RPT = _NP // _NS
_NBUF = 5
_L = 16

_SELU_A = 1.6732632423543772
_SELU_S = 1.0507009873554805

_mesh = functools.partial(
    plsc.VectorSubcoreMesh,
    core_axis_name="c", subcore_axis_name="s",
    num_cores=_NC, num_subcores=_NS)

_sc_params = pltpu.CompilerParams(
    use_tc_tiling_on_sc=False, needs_layout_passes=False)


def _rsqrt16(x):
  i = plsc.bitcast(x, jnp.int32)
  y = plsc.bitcast(0x5F3759DF - (i >> 1), jnp.float32)
  for _ in range(3):
    y = y * (1.5 - 0.5 * x * y * y)
  return jnp.where(x > 0, y, 0.0)


def _zero_fill(buf, nrows):
  z = jnp.zeros((_L,), jnp.float32)
  def body(i, carry):
    buf[i, :] = z
    return carry
  lax.fori_loop(0, nrows, body, 0)


def _scratch(nnode_bufs):
  return [
      [pltpu.VMEM((_NCH, _CHUNK), jnp.int32) for _ in range(2)],
      [pltpu.VMEM((_CHUNK, _L), jnp.float32) for _ in range(_NBUF)],
      [pltpu.VMEM((_RPT, _L), jnp.float32) for _ in range(nnode_bufs)],
      pltpu.VMEM_SHARED((_NP, _L), jnp.float32),
      [pltpu.SemaphoreType.DMA for _ in range(_NBUF)],
      [pltpu.SemaphoreType.DMA for _ in range(_NBUF)],
  ]


def _edge_phase(hs_hbm, out_hbm, src_v, dst_v, rows_v, sems, sems_s, agg_sh,
                cid, sid):
  for b in range(_NBUF - 1):
    pltpu.async_copy(hs_hbm.at[src_v.at[b]], rows_v[b], sems[b])
  def outer(i, carry):
    jj = i * _NBUF
    for b in range(_NBUF):
      j = jj + b
      pltpu.make_async_copy(
          hs_hbm.at[src_v.at[j]], rows_v[b], sems[b]).wait()
      pltpu.async_copy(rows_v[b], agg_sh.at[dst_v.at[j]], sems_s[b],
                       add=True)
      nxt = j + _NBUF - 1
      bn = (b + _NBUF - 1) % _NBUF
      @pl.when(j >= 1)
      def _():
        pltpu.make_async_copy(
            rows_v[bn], agg_sh.at[dst_v.at[j - 1]], sems_s[bn]).wait()
      @pl.when(nxt < _NCH)
      def _():
        pltpu.async_copy(hs_hbm.at[src_v.at[nxt]], rows_v[bn], sems[bn])
    return carry
  lax.fori_loop(0, _NCH // _NBUF, outer, 0)
  pltpu.make_async_copy(
      rows_v[(_NCH - 1) % _NBUF],
      agg_sh.at[dst_v.at[_NCH - 1]],
      sems_s[(_NCH - 1) % _NBUF]).wait()
  plsc.subcore_barrier()
  pltpu.sync_copy(agg_sh.at[pl.ds(sid * _RPT, _RPT)],
                  out_hbm.at[pl.ds(cid * _NP + sid * _RPT, _RPT)])


_PART = jax.ShapeDtypeStruct((_NC * _NP, _L), jnp.float32)
_NODE = jax.ShapeDtypeStruct((_NP, _L), jnp.float32)


def _sc_prop1(h0, src2d, dst2d):
  @functools.partial(
      pl.kernel,
      out_type=(_PART, _NODE, _NODE),
      mesh=_mesh(),
      compiler_params=_sc_params,
      scratch_types=_scratch(3) + [
          pltpu.VMEM((_NCH, _CHUNK), jnp.int32),
          pltpu.VMEM((_NP,), jnp.float32),
          pltpu.VMEM((_RPT,), jnp.float32),
          pltpu.VMEM((_RPT,), jnp.float32),
          pltpu.VMEM_SHARED((_NS * _NP,), jnp.float32),
      ],
  )
  def k(h0_hbm, src_hbm, dst_hbm, out_hbm, dinv_hbm, hs_hbm,
        idx, rows_v, nb, agg_sh, sems, sems_s, dst2_v, degl, degv, degt,
        deg_sh):
    cid = lax.axis_index("c")
    sid = lax.axis_index("s")
    wid = cid * _NS + sid
    wid2 = (1 - cid) * _NS + sid
    src_v, dst_v = idx
    h0v, dv, spare = nb
    pltpu.sync_copy(src_hbm.at[wid], src_v)
    pltpu.sync_copy(dst_hbm.at[wid], dst_v)
    pltpu.sync_copy(dst_hbm.at[wid2], dst2_v)
    r0 = sid * _RPT
    z = jnp.zeros((_L,), jnp.float32)
    def zrow(i, carry):
      degl[pl.ds(i * _L, _L)] = z
      return carry
    lax.fori_loop(0, _NP // _L, zrow, 0)
    _zero_fill(dv, _RPT)
    pltpu.sync_copy(dv, agg_sh.at[pl.ds(r0, _RPT)])
    ones = jnp.ones((_L,), jnp.float32)
    def hist(dref):
      def body(j, carry):
        for kk in range(_CHUNK // _L):
          plsc.addupdate_scatter(degl, [dref[j, pl.ds(kk * _L, _L)]], ones)
        return carry
      lax.fori_loop(0, _NCH, body, 0)
    hist(dst_v)
    hist(dst2_v)
    pltpu.sync_copy(degl, deg_sh.at[pl.ds(sid * _NP, _NP)])
    plsc.subcore_barrier()
    pltpu.sync_copy(deg_sh.at[pl.ds(r0, _RPT)], degv)
    for t in range(1, _NS):
      pltpu.sync_copy(deg_sh.at[pl.ds(t * _NP + r0, _RPT)], degt)
      def addt(i, carry):
        s = pl.ds(i * _L, _L)
        degv[s] = degv[s] + degt[s]
        return carry
      lax.fori_loop(0, _RPT // _L, addt, 0)
    pltpu.sync_copy(h0_hbm.at[pl.ds(r0, _RPT)], h0v)
    def row(r, carry):
      dvr = _rsqrt16(plsc.load_gather(degv, [jnp.full((_L,), r, jnp.int32)]))
      dv[r, :] = dvr
      h0v[r, :] = dvr * h0v[r, :]
      return carry
    lax.fori_loop(0, _RPT, row, 0)
    pltpu.sync_copy(dv, dinv_hbm.at[pl.ds(r0, _RPT)])
    pltpu.sync_copy(h0v, hs_hbm.at[pl.ds(r0, _RPT)])
    plsc.subcore_barrier()
    _edge_phase(hs_hbm, out_hbm, src_v, dst_v, rows_v, sems, sems_s,
                agg_sh, cid, sid)

  return k(h0, src2d, dst2d)


def _sc_prop2(p1, h0, dinv, src2d, dst2d):
  @functools.partial(
      pl.kernel,
      out_type=(_PART, _NODE, _NODE),
      mesh=_mesh(),
      compiler_params=_sc_params,
      scratch_types=_scratch(4),
  )
  def k(p1_hbm, h0_hbm, dinv_hbm, src_hbm, dst_hbm, out_hbm, h1_hbm, hs_hbm,
        idx, rows_v, nb, agg_sh, sems, sems_s):
    cid = lax.axis_index("c")
    sid = lax.axis_index("s")
    wid = cid * _NS + sid
    src_v, dst_v = idx
    pa, pb, h0v, dv = nb
    pltpu.sync_copy(src_hbm.at[wid], src_v)
    pltpu.sync_copy(dst_hbm.at[wid], dst_v)
    r0 = sid * _RPT
    _zero_fill(dv, _RPT)
    pltpu.sync_copy(dv, agg_sh.at[pl.ds(r0, _RPT)])
    pltpu.sync_copy(p1_hbm.at[pl.ds(r0, _RPT)], pa)
    pltpu.sync_copy(p1_hbm.at[pl.ds(_NP + r0, _RPT)], pb)
    pltpu.sync_copy(h0_hbm.at[pl.ds(r0, _RPT)], h0v)
    pltpu.sync_copy(dinv_hbm.at[pl.ds(r0, _RPT)], dv)
    def row(r, carry):
      dvr = dv[r, :]
      h1r = dvr * (pa[r, :] + pb[r, :]) + h0v[r, :]
      pa[r, :] = h1r
      pb[r, :] = dvr * h1r
      return carry
    lax.fori_loop(0, _RPT, row, 0)
    pltpu.sync_copy(pa, h1_hbm.at[pl.ds(r0, _RPT)])
    pltpu.sync_copy(pb, hs_hbm.at[pl.ds(r0, _RPT)])
    plsc.subcore_barrier()
    _edge_phase(hs_hbm, out_hbm, src_v, dst_v, rows_v, sems, sems_s,
                agg_sh, cid, sid)

  return k(p1, h0, dinv, src2d, dst2d)


def _sc_prop3(p2, h1, dinv, bnp, src2d, dst2d):
  @functools.partial(
      pl.kernel,
      out_type=(_PART, _NODE, _NODE),
      mesh=_mesh(),
      compiler_params=_sc_params,
      scratch_types=_scratch(4) + [
          pltpu.VMEM((8, _L), jnp.float32),
          pltpu.VMEM((2 * _NS, _L), jnp.float32),
          pltpu.VMEM_SHARED((2 * _NS, _L), jnp.float32),
      ],
  )
  def k(p2_hbm, h1_hbm, dinv_hbm, bnp_hbm, src_hbm, dst_hbm,
        out_hbm, g_hbm, hs_hbm,
        idx, rows_v, nb, agg_sh, sems, sems_s, bnv, stf, stats_sh):
    cid = lax.axis_index("c")
    sid = lax.axis_index("s")
    wid = cid * _NS + sid
    src_v, dst_v = idx
    pa, pb, h1v, dv = nb
    pltpu.sync_copy(src_hbm.at[wid], src_v)
    pltpu.sync_copy(dst_hbm.at[wid], dst_v)
    r0 = sid * _RPT
    _zero_fill(dv, _RPT)
    pltpu.sync_copy(dv, agg_sh.at[pl.ds(r0, _RPT)])
    pltpu.sync_copy(p2_hbm.at[pl.ds(r0, _RPT)], pa)
    pltpu.sync_copy(p2_hbm.at[pl.ds(_NP + r0, _RPT)], pb)
    pltpu.sync_copy(h1_hbm.at[pl.ds(r0, _RPT)], h1v)
    pltpu.sync_copy(dinv_hbm.at[pl.ds(r0, _RPT)], dv)
    pltpu.sync_copy(bnp_hbm, bnv)
    zero = jnp.zeros((_L,), jnp.float32)
    def row1(r, carry):
      acc, acc2 = carry
      h2r = dv[r, :] * (pa[r, :] + pb[r, :]) + h1v[r, :] + bnv[0, :]
      pa[r, :] = h2r
      sel = jnp.where(r0 + r < _N, h2r, zero)
      return acc + sel, acc2 + sel * sel
    acc, acc2 = lax.fori_loop(0, _RPT, row1, (zero, zero))
    stf[0, :] = acc
    stf[1, :] = acc2
    pltpu.sync_copy(stf.at[pl.ds(0, 2)], stats_sh.at[pl.ds(sid * 2, 2)])
    plsc.subcore_barrier()
    pltpu.sync_copy(stats_sh, stf)
    s = stf[0, :] + stf[2, :]
    s2 = stf[1, :] + stf[3, :]
    for t in range(2, _NS):
      s = s + stf[2 * t, :]
      s2 = s2 + stf[2 * t + 1, :]
    inv_n = jnp.float32(1.0 / _N)
    mean = s * inv_n
    var = s2 * inv_n - mean * mean
    istd = _rsqrt16(var + 1e-5)
    gam = istd * bnv[1, :]
    bet = bnv[2, :]
    def row2(r, carry):
      hn = (pa[r, :] - mean) * gam + bet
      g = _SELU_S * jnp.where(hn > 0, hn, _SELU_A * (jnp.exp(hn) - 1.0))
      pa[r, :] = g
      pb[r, :] = dv[r, :] * g
      return carry
    lax.fori_loop(0, _RPT, row2, 0)
    pltpu.sync_copy(pa, g_hbm.at[pl.ds(r0, _RPT)])
    pltpu.sync_copy(pb, hs_hbm.at[pl.ds(r0, _RPT)])
    plsc.subcore_barrier()
    _edge_phase(hs_hbm, out_hbm, src_v, dst_v, rows_v, sems, sems_s,
                agg_sh, cid, sid)

  return k(p2, h1, dinv, bnp, src2d, dst2d)


def _tc_a(x, W1):
  def body(x_ref, w_ref, o_ref):
    o_ref[0:_N, :] = jnp.dot(x_ref[...], w_ref[...],
                             preferred_element_type=jnp.float32)
    o_ref[_N:, :] = jnp.zeros((_NP - _N, _L), jnp.float32)
  return pl.pallas_call(body, out_shape=_NODE)(x, W1)


def _tc_b(p3, g, dinv, W2, b2):
  def body(p_ref, g_ref, dinv_ref, w2_ref, b2_ref, out_ref):
    agg = p_ref[0:_N, :] + p_ref[_NP:_NP + _N, :]
    h = dinv_ref[0:_N, :] * agg + g_ref[0:_N, :]
    logits = jnp.dot(h, w2_ref[...], preferred_element_type=jnp.float32)
    logits = logits + b2_ref[...]
    m = jnp.max(logits, axis=1, keepdims=True)
    e = jnp.exp(logits - m)
    out_ref[...] = e / jnp.sum(e, axis=1, keepdims=True)
  return pl.pallas_call(
      body, out_shape=jax.ShapeDtypeStruct((_N, _C), jnp.float32))(
          p3, g, dinv, W2, b2.reshape(1, _C))


def kernel(x, edge_index, W1, b1, gamma, beta, W2, b2):
  pad = jnp.full((_EPAD - _E,), _PADN, jnp.int32)
  src2d = jnp.concatenate([edge_index[0], pad]).reshape(_NW, _NCH, _CHUNK)
  dst2d = jnp.concatenate([edge_index[1], pad]).reshape(_NW, _NCH, _CHUNK)
  bnp = jnp.concatenate(
      [b1.reshape(1, _L), gamma.reshape(1, _L), beta.reshape(1, _L),
       jnp.zeros((5, _L), jnp.float32)], axis=0)

  h0 = _tc_a(x, W1)
  p1, dinv, _ = _sc_prop1(h0, src2d, dst2d)
  p2, h1, _ = _sc_prop2(p1, h0, dinv, src2d, dst2d)
  p3, g, _ = _sc_prop3(p2, h1, dinv, bnp, src2d, dst2d)
  return _tc_b(p3, g, dinv, W2, b2)


# speedup vs baseline: 1.8961x; 1.8961x over previous
import functools

import jax
import jax.numpy as jnp
from jax import lax
from jax.experimental import pallas as pl
from jax.experimental.pallas import tpu as pltpu
from jax.experimental.pallas import tpu_sc as plsc

_N, _E, _D, _H, _C = 10000, 320000, 128, 16, 16
_NC, _NS = 2, 16
_NW = _NC * _NS
_EPT = _E // _NW
_CHUNK = 80
_NCH = _EPT // _CHUNK
_NP = 10240
_RPT = _NP // _NS
_NBUF = 5
_L = 16

_SELU_A = 1.6732632423543772
_SELU_S = 1.0507009873554805

_mesh = functools.partial(
    plsc.VectorSubcoreMesh,
    core_axis_name="c", subcore_axis_name="s",
    num_cores=_NC, num_subcores=_NS)

_sc_params = pltpu.CompilerParams(
    use_tc_tiling_on_sc=False, needs_layout_passes=False)


def _rsqrt16(x):
  i = plsc.bitcast(x, jnp.int32)
  y = plsc.bitcast(0x5F3759DF - (i >> 1), jnp.float32)
  for _ in range(3):
    y = y * (1.5 - 0.5 * x * y * y)
  return jnp.where(x > 0, y, 0.0)


def _zero_fill(buf, nrows):
  z = jnp.zeros((_L,), jnp.float32)
  def body(i, carry):
    for u in range(16):
      buf[i * 16 + u, :] = z
    return carry
  lax.fori_loop(0, nrows // 16, body, 0)


def _scratch(nnode_bufs):
  return [
      [pltpu.VMEM((_NCH, _CHUNK), jnp.int32) for _ in range(2)],
      [pltpu.VMEM((_CHUNK, _L), jnp.float32) for _ in range(_NBUF)],
      [pltpu.VMEM((_RPT, _L), jnp.float32) for _ in range(nnode_bufs)],
      pltpu.VMEM_SHARED((_NP, _L), jnp.float32),
      [pltpu.SemaphoreType.DMA for _ in range(_NBUF)],
      [pltpu.SemaphoreType.DMA for _ in range(_NBUF)],
  ]


def _edge_phase(hs_hbm, out_hbm, src_v, dst_v, rows_v, sems, sems_s, agg_sh,
                cid, sid):
  for b in range(_NBUF - 1):
    pltpu.async_copy(hs_hbm.at[src_v.at[b]], rows_v[b], sems[b])
  def outer(i, carry):
    jj = i * _NBUF
    for b in range(_NBUF):
      j = jj + b
      pltpu.make_async_copy(
          hs_hbm.at[src_v.at[j]], rows_v[b], sems[b]).wait()
      pltpu.async_copy(rows_v[b], agg_sh.at[dst_v.at[j]], sems_s[b],
                       add=True)
      nxt = j + _NBUF - 1
      bn = (b + _NBUF - 1) % _NBUF
      @pl.when(j >= 1)
      def _():
        pltpu.make_async_copy(
            rows_v[bn], agg_sh.at[dst_v.at[j - 1]], sems_s[bn]).wait()
      @pl.when(nxt < _NCH)
      def _():
        pltpu.async_copy(hs_hbm.at[src_v.at[nxt]], rows_v[bn], sems[bn])
    return carry
  lax.fori_loop(0, _NCH // _NBUF, outer, 0)
  pltpu.make_async_copy(
      rows_v[(_NCH - 1) % _NBUF],
      agg_sh.at[dst_v.at[_NCH - 1]],
      sems_s[(_NCH - 1) % _NBUF]).wait()
  plsc.subcore_barrier()
  pltpu.sync_copy(agg_sh.at[pl.ds(sid * _RPT, _RPT)],
                  out_hbm.at[pl.ds(cid * _NP + sid * _RPT, _RPT)])


_PART = jax.ShapeDtypeStruct((_NC * _NP, _L), jnp.float32)
_NODE = jax.ShapeDtypeStruct((_NP, _L), jnp.float32)


def _sc_deg(dst2d):
  @functools.partial(
      pl.kernel,
      out_type=_PART,
      mesh=_mesh(),
      compiler_params=_sc_params,
      scratch_types=[
          pltpu.VMEM((_NCH, _CHUNK), jnp.int32),
          pltpu.VMEM((_CHUNK, _L), jnp.float32),
          pltpu.VMEM((_RPT, _L), jnp.float32),
          pltpu.VMEM_SHARED((_NP, _L), jnp.float32),
          pltpu.SemaphoreType.DMA,
      ],
  )
  def k(dst_hbm, out_hbm, dst_v, ones_v, stage_v, agg_sh, sem):
    cid = lax.axis_index("c")
    sid = lax.axis_index("s")
    wid = cid * _NS + sid
    pltpu.sync_copy(dst_hbm.at[wid], dst_v)
    one = jnp.ones((_L,), jnp.float32)
    def fill(i, carry):
      ones_v[i, :] = one
      return carry
    lax.fori_loop(0, _CHUNK, fill, 0)
    _zero_fill(stage_v, _RPT)
    pltpu.sync_copy(stage_v, agg_sh.at[pl.ds(sid * _RPT, _RPT)])
    plsc.subcore_barrier()
    def chunk(j, carry):
      pltpu.async_copy(ones_v, agg_sh.at[dst_v.at[j]], sem, add=True)
      @pl.when(j >= 4)
      def _():
        pltpu.make_async_copy(ones_v, agg_sh.at[dst_v.at[j - 4]], sem).wait()
      return carry
    lax.fori_loop(0, _NCH, chunk, 0)
    for r in range(4):
      pltpu.make_async_copy(
          ones_v, agg_sh.at[dst_v.at[_NCH - 4 + r]], sem).wait()
    plsc.subcore_barrier()
    pltpu.sync_copy(agg_sh.at[pl.ds(sid * _RPT, _RPT)],
                    out_hbm.at[pl.ds(cid * _NP + sid * _RPT, _RPT)])

  return k(dst2d)


def _sc_prop1(h0, degp, src2d, dst2d):
  @functools.partial(
      pl.kernel,
      out_type=(_PART, _NODE, _NODE),
      mesh=_mesh(),
      compiler_params=_sc_params,
      scratch_types=_scratch(4),
  )
  def k(h0_hbm, degp_hbm, src_hbm, dst_hbm, out_hbm, dinv_hbm, hs_hbm,
        idx, rows_v, nb, agg_sh, sems, sems_s):
    cid = lax.axis_index("c")
    sid = lax.axis_index("s")
    wid = cid * _NS + sid
    src_v, dst_v = idx
    pa, pb, h0v, dv = nb
    pltpu.sync_copy(src_hbm.at[wid], src_v)
    pltpu.sync_copy(dst_hbm.at[wid], dst_v)
    r0 = sid * _RPT
    _zero_fill(dv, _RPT)
    pltpu.sync_copy(dv, agg_sh.at[pl.ds(r0, _RPT)])
    pltpu.sync_copy(degp_hbm.at[pl.ds(r0, _RPT)], pa)
    pltpu.sync_copy(degp_hbm.at[pl.ds(_NP + r0, _RPT)], pb)
    pltpu.sync_copy(h0_hbm.at[pl.ds(r0, _RPT)], h0v)
    def row(rr, carry):
      for u in range(16):
        r = rr * 16 + u
        dvr = _rsqrt16(pa[r, :] + pb[r, :])
        dv[r, :] = dvr
        h0v[r, :] = dvr * h0v[r, :]
      return carry
    lax.fori_loop(0, _RPT // 16, row, 0)
    pltpu.sync_copy(dv, dinv_hbm.at[pl.ds(r0, _RPT)])
    pltpu.sync_copy(h0v, hs_hbm.at[pl.ds(r0, _RPT)])
    plsc.subcore_barrier()
    _edge_phase(hs_hbm, out_hbm, src_v, dst_v, rows_v, sems, sems_s,
                agg_sh, cid, sid)

  return k(h0, degp, src2d, dst2d)


def _sc_prop2(p1, h0, dinv, src2d, dst2d):
  @functools.partial(
      pl.kernel,
      out_type=(_PART, _NODE, _NODE),
      mesh=_mesh(),
      compiler_params=_sc_params,
      scratch_types=_scratch(4),
  )
  def k(p1_hbm, h0_hbm, dinv_hbm, src_hbm, dst_hbm, out_hbm, h1_hbm, hs_hbm,
        idx, rows_v, nb, agg_sh, sems, sems_s):
    cid = lax.axis_index("c")
    sid = lax.axis_index("s")
    wid = cid * _NS + sid
    src_v, dst_v = idx
    pa, pb, h0v, dv = nb
    pltpu.sync_copy(src_hbm.at[wid], src_v)
    pltpu.sync_copy(dst_hbm.at[wid], dst_v)
    r0 = sid * _RPT
    _zero_fill(dv, _RPT)
    pltpu.sync_copy(dv, agg_sh.at[pl.ds(r0, _RPT)])
    pltpu.sync_copy(p1_hbm.at[pl.ds(r0, _RPT)], pa)
    pltpu.sync_copy(p1_hbm.at[pl.ds(_NP + r0, _RPT)], pb)
    pltpu.sync_copy(h0_hbm.at[pl.ds(r0, _RPT)], h0v)
    pltpu.sync_copy(dinv_hbm.at[pl.ds(r0, _RPT)], dv)
    def row(rr, carry):
      for u in range(16):
        r = rr * 16 + u
        dvr = dv[r, :]
        h1r = dvr * (pa[r, :] + pb[r, :]) + h0v[r, :]
        pa[r, :] = h1r
        pb[r, :] = dvr * h1r
      return carry
    lax.fori_loop(0, _RPT // 16, row, 0)
    pltpu.sync_copy(pa, h1_hbm.at[pl.ds(r0, _RPT)])
    pltpu.sync_copy(pb, hs_hbm.at[pl.ds(r0, _RPT)])
    plsc.subcore_barrier()
    _edge_phase(hs_hbm, out_hbm, src_v, dst_v, rows_v, sems, sems_s,
                agg_sh, cid, sid)

  return k(p1, h0, dinv, src2d, dst2d)


def _sc_prop3(p2, h1, dinv, bnp, src2d, dst2d):
  @functools.partial(
      pl.kernel,
      out_type=(_PART, _NODE, _NODE),
      mesh=_mesh(),
      compiler_params=_sc_params,
      scratch_types=_scratch(4) + [
          pltpu.VMEM((8, _L), jnp.float32),
          pltpu.VMEM((2 * _NS, _L), jnp.float32),
          pltpu.VMEM_SHARED((2 * _NS, _L), jnp.float32),
      ],
  )
  def k(p2_hbm, h1_hbm, dinv_hbm, bnp_hbm, src_hbm, dst_hbm,
        out_hbm, g_hbm, hs_hbm,
        idx, rows_v, nb, agg_sh, sems, sems_s, bnv, stf, stats_sh):
    cid = lax.axis_index("c")
    sid = lax.axis_index("s")
    wid = cid * _NS + sid
    src_v, dst_v = idx
    pa, pb, h1v, dv = nb
    pltpu.sync_copy(src_hbm.at[wid], src_v)
    pltpu.sync_copy(dst_hbm.at[wid], dst_v)
    r0 = sid * _RPT
    _zero_fill(dv, _RPT)
    pltpu.sync_copy(dv, agg_sh.at[pl.ds(r0, _RPT)])
    pltpu.sync_copy(p2_hbm.at[pl.ds(r0, _RPT)], pa)
    pltpu.sync_copy(p2_hbm.at[pl.ds(_NP + r0, _RPT)], pb)
    pltpu.sync_copy(h1_hbm.at[pl.ds(r0, _RPT)], h1v)
    pltpu.sync_copy(dinv_hbm.at[pl.ds(r0, _RPT)], dv)
    pltpu.sync_copy(bnp_hbm, bnv)
    zero = jnp.zeros((_L,), jnp.float32)
    def row1(rr, carry):
      acc, acc2 = carry
      for u in range(16):
        r = rr * 16 + u
        h2r = dv[r, :] * (pa[r, :] + pb[r, :]) + h1v[r, :] + bnv[0, :]
        pa[r, :] = h2r
        sel = jnp.where(r0 + r < _N, h2r, zero)
        acc = acc + sel
        acc2 = acc2 + sel * sel
      return acc, acc2
    acc, acc2 = lax.fori_loop(0, _RPT // 16, row1, (zero, zero))
    stf[0, :] = acc
    stf[1, :] = acc2
    pltpu.sync_copy(stf.at[pl.ds(0, 2)], stats_sh.at[pl.ds(sid * 2, 2)])
    plsc.subcore_barrier()
    pltpu.sync_copy(stats_sh, stf)
    s = stf[0, :] + stf[2, :]
    s2 = stf[1, :] + stf[3, :]
    for t in range(2, _NS):
      s = s + stf[2 * t, :]
      s2 = s2 + stf[2 * t + 1, :]
    inv_n = jnp.float32(1.0 / _N)
    mean = s * inv_n
    var = s2 * inv_n - mean * mean
    istd = _rsqrt16(var + 1e-5)
    gam = istd * bnv[1, :]
    bet = bnv[2, :]
    def row2(rr, carry):
      for u in range(16):
        r = rr * 16 + u
        hn = (pa[r, :] - mean) * gam + bet
        g = _SELU_S * jnp.where(hn > 0, hn, _SELU_A * (jnp.exp(hn) - 1.0))
        pa[r, :] = g
        pb[r, :] = dv[r, :] * g
      return carry
    lax.fori_loop(0, _RPT // 16, row2, 0)
    pltpu.sync_copy(pa, g_hbm.at[pl.ds(r0, _RPT)])
    pltpu.sync_copy(pb, hs_hbm.at[pl.ds(r0, _RPT)])
    plsc.subcore_barrier()
    _edge_phase(hs_hbm, out_hbm, src_v, dst_v, rows_v, sems, sems_s,
                agg_sh, cid, sid)

  return k(p2, h1, dinv, bnp, src2d, dst2d)


def _tc_a(x, W1):
  def body(x_ref, w_ref, o_ref):
    o_ref[0:_N, :] = jnp.dot(x_ref[...], w_ref[...],
                             preferred_element_type=jnp.float32)
    o_ref[_N:, :] = jnp.zeros((_NP - _N, _L), jnp.float32)
  return pl.pallas_call(body, out_shape=_NODE)(x, W1)


def _tc_b(p3, g, dinv, W2, b2):
  def body(p_ref, g_ref, dinv_ref, w2_ref, b2_ref, out_ref):
    agg = p_ref[0:_N, :] + p_ref[_NP:_NP + _N, :]
    h = dinv_ref[0:_N, :] * agg + g_ref[0:_N, :]
    logits = jnp.dot(h, w2_ref[...], preferred_element_type=jnp.float32)
    logits = logits + b2_ref[...]
    m = jnp.max(logits, axis=1, keepdims=True)
    e = jnp.exp(logits - m)
    out_ref[...] = e / jnp.sum(e, axis=1, keepdims=True)
  return pl.pallas_call(
      body, out_shape=jax.ShapeDtypeStruct((_N, _C), jnp.float32))(
          p3, g, dinv, W2, b2.reshape(1, _C))


def kernel(x, edge_index, W1, b1, gamma, beta, W2, b2):
  src2d = edge_index[0].reshape(_NW, _NCH, _CHUNK)
  dst2d = edge_index[1].reshape(_NW, _NCH, _CHUNK)
  bnp = jnp.concatenate(
      [b1.reshape(1, _L), gamma.reshape(1, _L), beta.reshape(1, _L),
       jnp.zeros((5, _L), jnp.float32)], axis=0)

  h0 = _tc_a(x, W1)
  degp = _sc_deg(dst2d)
  p1, dinv, _ = _sc_prop1(h0, degp, src2d, dst2d)
  p2, h1, _ = _sc_prop2(p1, h0, dinv, src2d, dst2d)
  p3, g, _ = _sc_prop3(p2, h1, dinv, bnp, src2d, dst2d)
  return _tc_b(p3, g, dinv, W2, b2)
